# baseline (speedup 1.0000x reference)
import functools

import jax
import jax.numpy as jnp
from jax import lax
from jax.experimental import pallas as pl
from jax.experimental.pallas import tpu as pltpu
from jax.experimental.pallas import tpu_sc as plsc

_VOCAB = 1000000
_D = 32
_B = 16384
_CTX = 20
_NEG = 20

_NC = 2
_NS = 16
_NW = _NC * _NS
_L = 16

_BPW = _B // _NW
_C = 32
_NCHUNK = _BPW // _C
_GPC = _C // _L
_RPC = _C * _CTX
_NJ = _RPC // _L
_IDXROWS = _RPC // 128

_MESH = dict(core_axis_name="c", subcore_axis_name="s",
             num_cores=_NC, num_subcores=_NS)
_PARAMS = dict(use_tc_tiling_on_sc=False, needs_layout_passes=False)


def _flatten_coords(iota):
    items = [(j * _L + iota) // _CTX for j in range(_NJ)]
    slots = [(j * _L + iota) % _CTX for j in range(_NJ)]
    return items, slots


def _sc_a_body(ctx_idx_hbm, win_hbm, ctxm_out,
               ctx_idx2d, ctx_idx_v, ctx_rows, ctx_f32, ctxm_acc,
               sem0, sem1):
    wid = lax.axis_index("s") * _NC + lax.axis_index("c")
    iota = lax.iota(jnp.int32, _L)
    pltpu.sync_copy(ctx_idx_hbm.at[pl.ds(wid * _BPW, _BPW)], ctx_idx2d)
    flat_items, flat_slots = _flatten_coords(iota)
    sems = [sem0, sem1]

    def fire(c, s):
        for j in range(_NJ):
            ctx_idx_v[s, pl.ds(j * _L, _L)] = plsc.load_gather(
                ctx_idx2d, [flat_items[j] + c * _C, flat_slots[j]])
        for j in range(_IDXROWS):
            pltpu.async_copy(
                win_hbm.at[ctx_idx_v.at[s, pl.ds(j * 128, 128)]],
                ctx_rows.at[s, pl.ds(j * 128, 128)], sems[s])

    def drain(s):
        for j in range(_IDXROWS):
            pltpu.make_async_copy(
                win_hbm.at[ctx_idx_v.at[s, pl.ds(j * 128, 128)]],
                ctx_rows.at[s, pl.ds(j * 128, 128)], sems[s]).wait()

    def compute(c, s):
        def expand_body(r, _):
            for u in range(4):
                a, b = plsc.unpack(ctx_rows[s, r * 4 + u, :],
                                   format=plsc.PackFormat.INTERLEAVED)
                ctx_f32[r * 4 + u, pl.ds(0, _L)] = a
                ctx_f32[r * 4 + u, pl.ds(_L, _L)] = b
            return 0
        lax.fori_loop(0, _RPC // 4, expand_body, 0)

        def group_body(g, _):
            row0 = (g * _L + iota) * _CTX
            off = c * _C + g * _L
            for d in range(_D):
                col = jnp.full((_L,), d, jnp.int32)
                acc = plsc.load_gather(ctx_f32, [row0, col])
                for n in range(1, _CTX):
                    acc = acc + plsc.load_gather(ctx_f32, [row0 + n, col])
                ctxm_acc[d, pl.ds(off, _L)] = acc * (1.0 / _CTX)
            return 0
        lax.fori_loop(0, _GPC, group_body, 0)

    fire(0, 0)

    def super_body(k, _):
        c0 = 2 * k
        fire(c0 + 1, 1)
        drain(0)
        compute(c0, 0)

        @pl.when(c0 + 2 < _NCHUNK)
        def _():
            fire(c0 + 2, 0)

        drain(1)
        compute(c0 + 1, 1)
        return 0

    lax.fori_loop(0, _NCHUNK // 2, super_body, 0)
    pltpu.sync_copy(ctxm_acc, ctxm_out.at[wid])


def _sc_b_body(tgt_idx_hbm, neg_idx_hbm, wout_hbm, ctxm_hbm,
               pos_out, neg_out,
               neg_idx2d, neg_idx_v, tgt_idx_v, neg_rows, tgt_rows,
               neg_f32, tgt_f32, ctxm_v, pos_acc, neg_acc, sem0, sem1):
    wid = lax.axis_index("s") * _NC + lax.axis_index("c")
    iota = lax.iota(jnp.int32, _L)
    pltpu.sync_copy(neg_idx_hbm.at[pl.ds(wid * _BPW, _BPW)], neg_idx2d)
    pltpu.sync_copy(ctxm_hbm.at[wid], ctxm_v)
    pltpu.sync_copy(tgt_idx_hbm.at[pl.ds(wid * _BPW, _BPW)], tgt_idx_v)
    flat_items, flat_slots = _flatten_coords(iota)
    sems = [sem0, sem1]

    def fire(c, s):
        for j in range(_NJ):
            neg_idx_v[s, pl.ds(j * _L, _L)] = plsc.load_gather(
                neg_idx2d, [flat_items[j] + c * _C, flat_slots[j]])
        for j in range(_IDXROWS):
            pltpu.async_copy(
                wout_hbm.at[neg_idx_v.at[s, pl.ds(j * 128, 128)]],
                neg_rows.at[s, pl.ds(j * 128, 128)], sems[s])
        pltpu.async_copy(
            wout_hbm.at[tgt_idx_v.at[pl.ds(c * _C, _C)]],
            tgt_rows.at[s], sems[s])

    def drain(s):
        for j in range(_IDXROWS):
            pltpu.make_async_copy(
                wout_hbm.at[neg_idx_v.at[s, pl.ds(j * 128, 128)]],
                neg_rows.at[s, pl.ds(j * 128, 128)], sems[s]).wait()
        pltpu.make_async_copy(
            wout_hbm.at[tgt_idx_v.at[pl.ds(0, _C)]],
            tgt_rows.at[s], sems[s]).wait()

    def compute(c, s):
        def expand_body(r, _):
            for u in range(4):
                a, b = plsc.unpack(neg_rows[s, r * 4 + u, :],
                                   format=plsc.PackFormat.INTERLEAVED)
                neg_f32[r * 4 + u, pl.ds(0, _L)] = a
                neg_f32[r * 4 + u, pl.ds(_L, _L)] = b
            return 0
        lax.fori_loop(0, _RPC // 4, expand_body, 0)

        def texpand_body(r, _):
            a, b = plsc.unpack(tgt_rows[s, r, :],
                               format=plsc.PackFormat.INTERLEAVED)
            tgt_f32[r, pl.ds(0, _L)] = a
            tgt_f32[r, pl.ds(_L, _L)] = b
            return 0
        lax.fori_loop(0, _C, texpand_body, 0)

        def group_body(g, _):
            row0 = (g * _L + iota) * _CTX
            trow = g * _L + iota
            off = c * _C + g * _L
            pos = jnp.zeros((_L,), jnp.float32)
            negs = [jnp.zeros((_L,), jnp.float32) for _ in range(_NEG)]
            for d in range(_D):
                col = jnp.full((_L,), d, jnp.int32)
                cm = ctxm_v[d, pl.ds(off, _L)]
                pos = pos + plsc.load_gather(tgt_f32, [trow, col]) * cm
                for n in range(_NEG):
                    negs[n] = negs[n] + plsc.load_gather(
                        neg_f32, [row0 + n, col]) * cm
            pos_acc[pl.ds(off, _L)] = pos
            for n in range(_NEG):
                neg_acc[n, pl.ds(off, _L)] = negs[n]
            return 0
        lax.fori_loop(0, _GPC, group_body, 0)

    fire(0, 0)

    def super_body(k, _):
        c0 = 2 * k
        fire(c0 + 1, 1)
        drain(0)
        compute(c0, 0)

        @pl.when(c0 + 2 < _NCHUNK)
        def _():
            fire(c0 + 2, 0)

        drain(1)
        compute(c0 + 1, 1)
        return 0

    lax.fori_loop(0, _NCHUNK // 2, super_body, 0)
    pltpu.sync_copy(pos_acc, pos_out.at[pl.ds(wid * _BPW, _BPW)])
    pltpu.sync_copy(neg_acc, neg_out.at[wid])


def _sc_ctx_means(ctx_idx, w_in):
    mesh = plsc.VectorSubcoreMesh(**_MESH)
    return pl.kernel(
        _sc_a_body,
        out_type=jax.ShapeDtypeStruct((_NW, _D, _BPW), jnp.float32),
        mesh=mesh,
        compiler_params=pltpu.CompilerParams(**_PARAMS),
        scratch_types=[
            pltpu.VMEM((_BPW, _CTX), jnp.int32),
            pltpu.VMEM((2, _RPC), jnp.int32),
            pltpu.VMEM((2, _RPC, _D), jnp.bfloat16),
            pltpu.VMEM((_RPC, _D), jnp.float32),
            pltpu.VMEM((_D, _BPW), jnp.float32),
            pltpu.SemaphoreType.DMA,
            pltpu.SemaphoreType.DMA,
        ],
    )(ctx_idx, w_in)


def _sc_scores(tgt_idx, neg_idx, w_out, ctxm):
    mesh = plsc.VectorSubcoreMesh(**_MESH)
    return pl.kernel(
        _sc_b_body,
        out_type=[
            jax.ShapeDtypeStruct((_B,), jnp.float32),
            jax.ShapeDtypeStruct((_NW, _NEG, _BPW), jnp.float32),
        ],
        mesh=mesh,
        compiler_params=pltpu.CompilerParams(**_PARAMS),
        scratch_types=[
            pltpu.VMEM((_BPW, _NEG), jnp.int32),
            pltpu.VMEM((2, _RPC), jnp.int32),
            pltpu.VMEM((_BPW,), jnp.int32),
            pltpu.VMEM((2, _RPC, _D), jnp.bfloat16),
            pltpu.VMEM((2, _C, _D), jnp.bfloat16),
            pltpu.VMEM((_RPC, _D), jnp.float32),
            pltpu.VMEM((_C, _D), jnp.float32),
            pltpu.VMEM((_D, _BPW), jnp.float32),
            pltpu.VMEM((_BPW,), jnp.float32),
            pltpu.VMEM((_NEG, _BPW), jnp.float32),
            pltpu.SemaphoreType.DMA,
            pltpu.SemaphoreType.DMA,
        ],
    )(tgt_idx, neg_idx, w_out, ctxm)


def _tc_loss_body(pos_ref, neg_ref, out_ref):
    pos = pos_ref[...]
    neg = neg_ref[...]
    lsp = jnp.sum(jnp.log(jax.nn.sigmoid(pos) + 1e-10))
    lsn = jnp.sum(jnp.log(jax.nn.sigmoid(-neg) + 1e-10))
    out_ref[0, 0] = -(lsp + lsn) / _B


@jax.jit
def kernel(context_words, target_words, negative_words, W_in, W_out):
    ctx2d = context_words.astype(jnp.int32)
    neg2d = negative_words.astype(jnp.int32)
    tgt = target_words.astype(jnp.int32)
    ctxm = _sc_ctx_means(ctx2d, W_in.astype(jnp.bfloat16))
    pos_sc, neg_sc = _sc_scores(tgt, neg2d, W_out.astype(jnp.bfloat16),
                                ctxm)
    loss = pl.pallas_call(
        _tc_loss_body,
        out_shape=jax.ShapeDtypeStruct((1, 1), jnp.float32),
        out_specs=pl.BlockSpec(memory_space=pltpu.SMEM),
    )(pos_sc.reshape(128, 128), neg_sc.reshape(_NW * _NEG, _BPW))
    return loss[0, 0]


# speedup vs baseline: 1.4048x; 1.4048x over previous
import functools

import jax
import jax.numpy as jnp
from jax import lax
from jax.experimental import pallas as pl
from jax.experimental.pallas import tpu as pltpu
from jax.experimental.pallas import tpu_sc as plsc

_VOCAB = 1000000
_D = 32
_B = 16384
_CTX = 20
_NEG = 20

_NC = 2
_NS = 16
_NW = _NC * _NS
_L = 16

_BPW = _B // _NW
_C = 32
_NCHUNK = _BPW // _C
_GPC = _C // _L
_RPC = _C * _CTX
_NJ = _RPC // _L
_IDXROWS = _RPC // 128

_MESH = dict(core_axis_name="c", subcore_axis_name="s",
             num_cores=_NC, num_subcores=_NS)
_PARAMS = dict(use_tc_tiling_on_sc=False, needs_layout_passes=False)


def _flatten_coords(iota):
    items = [(j * _L + iota) // _CTX for j in range(_NJ)]
    slots = [(j * _L + iota) % _CTX for j in range(_NJ)]
    return items, slots


def _sc_a_body(ctx_idx_hbm, win_hbm, ctxm_out,
               ctx_idx2d, ctx_idx_v, ctx_rows, ctxm_acc, sem0, sem1):
    wid = lax.axis_index("s") * _NC + lax.axis_index("c")
    iota = lax.iota(jnp.int32, _L)
    pltpu.sync_copy(ctx_idx_hbm.at[pl.ds(wid * _BPW, _BPW)], ctx_idx2d)
    flat_items, flat_slots = _flatten_coords(iota)
    sems = [sem0, sem1]

    def fire(c, s):
        for j in range(_NJ):
            ctx_idx_v[s, pl.ds(j * _L, _L)] = plsc.load_gather(
                ctx_idx2d, [flat_items[j] + c * _C, flat_slots[j]])
        for j in range(_IDXROWS):
            pltpu.async_copy(
                win_hbm.at[ctx_idx_v.at[s, pl.ds(j * 128, 128)]],
                ctx_rows.at[s, pl.ds(j * 128, 128)], sems[s])

    def drain(s):
        for j in range(_IDXROWS):
            pltpu.make_async_copy(
                win_hbm.at[ctx_idx_v.at[s, pl.ds(j * 128, 128)]],
                ctx_rows.at[s, pl.ds(j * 128, 128)], sems[s]).wait()

    def compute(c, s):
        def group_body(g, _):
            row0 = (g * _L + iota) * _CTX
            off = c * _C + g * _L
            for d in range(_D):
                col = jnp.full((_L,), d, jnp.int32)
                acc = plsc.load_gather(ctx_rows.at[s], [row0, col])
                for n in range(1, _CTX):
                    acc = acc + plsc.load_gather(ctx_rows.at[s],
                                                 [row0 + n, col])
                ctxm_acc[d, pl.ds(off, _L)] = acc * (1.0 / _CTX)
            return 0
        lax.fori_loop(0, _GPC, group_body, 0)

    fire(0, 0)

    def super_body(k, _):
        c0 = 2 * k
        fire(c0 + 1, 1)
        drain(0)
        compute(c0, 0)

        @pl.when(c0 + 2 < _NCHUNK)
        def _():
            fire(c0 + 2, 0)

        drain(1)
        compute(c0 + 1, 1)
        return 0

    lax.fori_loop(0, _NCHUNK // 2, super_body, 0)
    pltpu.sync_copy(ctxm_acc, ctxm_out.at[wid])


def _sc_b_body(tgt_idx_hbm, neg_idx_hbm, wout_hbm, ctxm_hbm,
               pos_out, neg_out,
               neg_idx2d, neg_idx_v, tgt_idx_v, neg_rows, tgt_rows,
               ctxm_v, pos_acc, neg_acc, sem0, sem1):
    wid = lax.axis_index("s") * _NC + lax.axis_index("c")
    iota = lax.iota(jnp.int32, _L)
    pltpu.sync_copy(neg_idx_hbm.at[pl.ds(wid * _BPW, _BPW)], neg_idx2d)
    pltpu.sync_copy(ctxm_hbm.at[wid], ctxm_v)
    pltpu.sync_copy(tgt_idx_hbm.at[pl.ds(wid * _BPW, _BPW)], tgt_idx_v)
    flat_items, flat_slots = _flatten_coords(iota)
    sems = [sem0, sem1]

    def fire(c, s):
        for j in range(_NJ):
            neg_idx_v[s, pl.ds(j * _L, _L)] = plsc.load_gather(
                neg_idx2d, [flat_items[j] + c * _C, flat_slots[j]])
        for j in range(_IDXROWS):
            pltpu.async_copy(
                wout_hbm.at[neg_idx_v.at[s, pl.ds(j * 128, 128)]],
                neg_rows.at[s, pl.ds(j * 128, 128)], sems[s])
        pltpu.async_copy(
            wout_hbm.at[tgt_idx_v.at[pl.ds(c * _C, _C)]],
            tgt_rows.at[s], sems[s])

    def drain(s):
        for j in range(_IDXROWS):
            pltpu.make_async_copy(
                wout_hbm.at[neg_idx_v.at[s, pl.ds(j * 128, 128)]],
                neg_rows.at[s, pl.ds(j * 128, 128)], sems[s]).wait()
        pltpu.make_async_copy(
            wout_hbm.at[tgt_idx_v.at[pl.ds(0, _C)]],
            tgt_rows.at[s], sems[s]).wait()

    def compute(c, s):
        def group_body(g, _):
            row0 = (g * _L + iota) * _CTX
            trow = g * _L + iota
            off = c * _C + g * _L
            pos = jnp.zeros((_L,), jnp.float32)
            negs = [jnp.zeros((_L,), jnp.float32) for _ in range(_NEG)]
            for d in range(_D):
                col = jnp.full((_L,), d, jnp.int32)
                cm = ctxm_v[d, pl.ds(off, _L)]
                pos = pos + plsc.load_gather(tgt_rows.at[s],
                                             [trow, col]) * cm
                for n in range(_NEG):
                    negs[n] = negs[n] + plsc.load_gather(
                        neg_rows.at[s], [row0 + n, col]) * cm
            pos_acc[pl.ds(off, _L)] = pos
            for n in range(_NEG):
                neg_acc[n, pl.ds(off, _L)] = negs[n]
            return 0
        lax.fori_loop(0, _GPC, group_body, 0)

    fire(0, 0)

    def super_body(k, _):
        c0 = 2 * k
        fire(c0 + 1, 1)
        drain(0)
        compute(c0, 0)

        @pl.when(c0 + 2 < _NCHUNK)
        def _():
            fire(c0 + 2, 0)

        drain(1)
        compute(c0 + 1, 1)
        return 0

    lax.fori_loop(0, _NCHUNK // 2, super_body, 0)
    pltpu.sync_copy(pos_acc, pos_out.at[pl.ds(wid * _BPW, _BPW)])
    pltpu.sync_copy(neg_acc, neg_out.at[wid])


def _sc_ctx_means(ctx_idx, w_in):
    mesh = plsc.VectorSubcoreMesh(**_MESH)
    return pl.kernel(
        _sc_a_body,
        out_type=jax.ShapeDtypeStruct((_NW, _D, _BPW), jnp.float32),
        mesh=mesh,
        compiler_params=pltpu.CompilerParams(**_PARAMS),
        scratch_types=[
            pltpu.VMEM((_BPW, _CTX), jnp.int32),
            pltpu.VMEM((2, _RPC), jnp.int32),
            pltpu.VMEM((2, _RPC, _D), jnp.float32),
            pltpu.VMEM((_D, _BPW), jnp.float32),
            pltpu.SemaphoreType.DMA,
            pltpu.SemaphoreType.DMA,
        ],
    )(ctx_idx, w_in)


def _sc_scores(tgt_idx, neg_idx, w_out, ctxm):
    mesh = plsc.VectorSubcoreMesh(**_MESH)
    return pl.kernel(
        _sc_b_body,
        out_type=[
            jax.ShapeDtypeStruct((_B,), jnp.float32),
            jax.ShapeDtypeStruct((_NW, _NEG, _BPW), jnp.float32),
        ],
        mesh=mesh,
        compiler_params=pltpu.CompilerParams(**_PARAMS),
        scratch_types=[
            pltpu.VMEM((_BPW, _NEG), jnp.int32),
            pltpu.VMEM((2, _RPC), jnp.int32),
            pltpu.VMEM((_BPW,), jnp.int32),
            pltpu.VMEM((2, _RPC, _D), jnp.float32),
            pltpu.VMEM((2, _C, _D), jnp.float32),
            pltpu.VMEM((_D, _BPW), jnp.float32),
            pltpu.VMEM((_BPW,), jnp.float32),
            pltpu.VMEM((_NEG, _BPW), jnp.float32),
            pltpu.SemaphoreType.DMA,
            pltpu.SemaphoreType.DMA,
        ],
    )(tgt_idx, neg_idx, w_out, ctxm)


def _tc_loss_body(pos_ref, neg_ref, out_ref):
    pos = pos_ref[...]
    neg = neg_ref[...]
    lsp = jnp.sum(jnp.log(jax.nn.sigmoid(pos) + 1e-10))
    lsn = jnp.sum(jnp.log(jax.nn.sigmoid(-neg) + 1e-10))
    out_ref[0, 0] = -(lsp + lsn) / _B


@jax.jit
def kernel(context_words, target_words, negative_words, W_in, W_out):
    ctx2d = context_words.astype(jnp.int32)
    neg2d = negative_words.astype(jnp.int32)
    tgt = target_words.astype(jnp.int32)
    ctxm = _sc_ctx_means(ctx2d, W_in)
    pos_sc, neg_sc = _sc_scores(tgt, neg2d, W_out, ctxm)
    loss = pl.pallas_call(
        _tc_loss_body,
        out_shape=jax.ShapeDtypeStruct((1, 1), jnp.float32),
        out_specs=pl.BlockSpec(memory_space=pltpu.SMEM),
    )(pos_sc.reshape(128, 128), neg_sc.reshape(_NW * _NEG, _BPW))
    return loss[0, 0]
